# baseline (speedup 1.0000x reference)
import jax
import jax.numpy as jnp
from jax import lax
from jax.experimental import pallas as pl
from jax.experimental.pallas import tpu as pltpu
from jax.experimental.pallas import tpu_sc as plsc

DIM = 1024
TILE_N = 2048
DEPTH = 3
NUM_TOKENS = 256
VOCAB = 100000
N_FULL = VOCAB // TILE_N
REM = VOCAB - N_FULL * TILE_N
ISSUE_LAST_AT = N_FULL - DEPTH
GW = 128
SPLIT = 8


def _sc_gather(idx2d, emb_table):
    mesh = plsc.VectorSubcoreMesh(
        core_axis_name="core", subcore_axis_name="subcore")
    n_sub = NUM_TOKENS * SPLIT
    sub_d = DIM // SPLIT

    @pl.kernel(
        out_type=jax.ShapeDtypeStruct((n_sub, sub_d), jnp.float32),
        mesh=mesh)
    def gk(i_hbm, x_hbm, o_hbm):
        def body(i_vmem, o_vmem):
            pltpu.sync_copy(x_hbm.at[i_vmem.at[0]], o_vmem)

        pltpu.emit_pipeline(
            body,
            grid=(n_sub // GW,),
            in_specs=[pl.BlockSpec((1, GW), lambda i: (0, i))],
            out_specs=[pl.BlockSpec((GW, sub_d), lambda i: (i, 0))],
            core_axis_name=("core", "subcore"),
            dimension_semantics=(pltpu.PARALLEL,),
        )(i_hbm, o_hbm)

    return gk(idx2d, emb_table.reshape(VOCAB * SPLIT, sub_d))


def _body(h_hbm, w_hbm, b_hbm, out_hbm, h_ref, wbuf, obuf,
          wbuf_last, obuf_last, b_ref, gsem, bsem, wsem, osem):
    def _w_fill(j_start, slot):
        return pltpu.make_async_copy(
            w_hbm.at[pl.ds(j_start, TILE_N), :], wbuf.at[slot], wsem.at[slot])

    def _w_fill_last():
        return pltpu.make_async_copy(
            w_hbm.at[pl.ds(N_FULL * TILE_N, REM), :], wbuf_last, gsem)

    def _out_copy(slot, col_start):
        return pltpu.make_async_copy(
            obuf.at[slot], out_hbm.at[:, pl.ds(col_start, TILE_N)],
            osem.at[slot])

    h_copy = pltpu.make_async_copy(h_hbm, h_ref, gsem)
    h_copy.start()
    b_copy = pltpu.make_async_copy(b_hbm, b_ref, bsem)
    b_copy.start()
    for j in range(DEPTH):
        _w_fill(j * TILE_N, j).start()
    h_copy.wait()
    b_copy.wait()
    h = h_ref[...]

    def tile_step(i, _):
        slot = lax.rem(i, DEPTH)
        oslot = lax.rem(i, 2)
        s = i * TILE_N
        _w_fill(s, slot).wait()
        acc = lax.dot_general(
            h, wbuf[slot],
            dimension_numbers=(((1,), (1,)), ((), ())),
            preferred_element_type=jnp.float32,
        )

        @pl.when(i >= 2)
        def _reclaim():
            _out_copy(oslot, (i - 2) * TILE_N).wait()

        obuf[oslot] = acc + b_ref[:, pl.ds(s, TILE_N)]
        _out_copy(oslot, s).start()

        @pl.when(i + DEPTH < N_FULL)
        def _refill():
            _w_fill((i + DEPTH) * TILE_N, slot).start()

        @pl.when(i == ISSUE_LAST_AT)
        def _refill_last():
            _w_fill_last().start()

        return 0

    lax.fori_loop(0, N_FULL, tile_step, 0, unroll=4)

    _w_fill_last().wait()
    acc = lax.dot_general(
        h, wbuf_last[...],
        dimension_numbers=(((1,), (1,)), ((), ())),
        preferred_element_type=jnp.float32,
    )
    oslot = N_FULL % 2
    _out_copy(oslot, (N_FULL - 2) * TILE_N).wait()
    obuf_last[...] = acc + b_ref[:, pl.ds(N_FULL * TILE_N, REM)]
    last_out = pltpu.make_async_copy(
        obuf_last, out_hbm.at[:, pl.ds(N_FULL * TILE_N, REM)], osem.at[oslot])
    last_out.start()

    _out_copy(1 - oslot, (N_FULL - 1) * TILE_N).wait()
    last_out.wait()


@jax.jit
def kernel(x, emb_table, W, b):
    B, L = x.shape
    T = B * L
    V, D = W.shape
    idx = x.reshape(T).astype(jnp.int32)
    sub = jnp.arange(SPLIT, dtype=jnp.int32)
    idx2d = (idx[:, None] * SPLIT + sub[None, :]).reshape(1, T * SPLIT)
    b2 = b.reshape(1, V)

    h = _sc_gather(idx2d, emb_table).reshape(T, D)

    logits = pl.pallas_call(
        _body,
        grid=(1,),
        in_specs=[
            pl.BlockSpec(memory_space=pl.ANY),
            pl.BlockSpec(memory_space=pl.ANY),
            pl.BlockSpec(memory_space=pl.ANY),
        ],
        out_specs=pl.BlockSpec(memory_space=pl.ANY),
        scratch_shapes=[
            pltpu.VMEM((T, D), jnp.float32),
            pltpu.VMEM((DEPTH, TILE_N, D), jnp.float32),
            pltpu.VMEM((2, T, TILE_N), jnp.float32),
            pltpu.VMEM((REM, D), jnp.float32),
            pltpu.VMEM((T, REM), jnp.float32),
            pltpu.VMEM((1, V), jnp.float32),
            pltpu.SemaphoreType.DMA,
            pltpu.SemaphoreType.DMA,
            pltpu.SemaphoreType.DMA((DEPTH,)),
            pltpu.SemaphoreType.DMA((2,)),
        ],
        out_shape=jax.ShapeDtypeStruct((T, V), jnp.float32),
        compiler_params=pltpu.CompilerParams(
            dimension_semantics=("arbitrary",),
        ),
    )(h, W, b2)

    return logits.reshape(B, L, V)


# speedup vs baseline: 3.5942x; 3.5942x over previous
import jax
import jax.numpy as jnp
from jax import lax
from jax.experimental import pallas as pl
from jax.experimental.pallas import tpu as pltpu

DIM = 1024
TILE_N = 2560
DEPTH = 3
NUM_TOKENS = 256
VOCAB = 100000
N_FULL = VOCAB // TILE_N
REM = VOCAB - N_FULL * TILE_N
ISSUE_LAST_AT = N_FULL - DEPTH


def _body(idx_ref, emb_hbm, w_hbm, b_hbm, out_hbm, h_ref, wbuf, obuf,
          wbuf_last, obuf_last, b_ref, gsem, bsem, wsem, osem):
    def _w_fill(j_start, slot):
        return pltpu.make_async_copy(
            w_hbm.at[pl.ds(j_start, TILE_N), :], wbuf.at[slot], wsem.at[slot])

    def _w_fill_last():
        return pltpu.make_async_copy(
            w_hbm.at[pl.ds(N_FULL * TILE_N, REM), :], wbuf_last, gsem)

    def _out_copy(slot, col_start):
        return pltpu.make_async_copy(
            obuf.at[slot], out_hbm.at[:, pl.ds(col_start, TILE_N)],
            osem.at[slot])

    def issue_row(t, _):
        pltpu.make_async_copy(
            emb_hbm.at[pl.ds(idx_ref[t], 1), :],
            h_ref.at[pl.ds(t, 1), :], gsem).start()
        return 0

    def wait_row(t, _):
        pltpu.make_async_copy(
            emb_hbm.at[pl.ds(idx_ref[t], 1), :],
            h_ref.at[pl.ds(t, 1), :], gsem).wait()
        return 0

    lax.fori_loop(0, NUM_TOKENS, issue_row, 0)
    b_copy = pltpu.make_async_copy(b_hbm, b_ref, bsem)
    b_copy.start()
    for j in range(DEPTH):
        _w_fill(j * TILE_N, j).start()
    lax.fori_loop(0, NUM_TOKENS, wait_row, 0)
    b_copy.wait()
    h = h_ref[...]

    def tile_step(i, _):
        slot = lax.rem(i, DEPTH)
        oslot = lax.rem(i, 2)
        s = i * TILE_N
        _w_fill(s, slot).wait()
        acc = lax.dot_general(
            h, wbuf[slot],
            dimension_numbers=(((1,), (1,)), ((), ())),
            preferred_element_type=jnp.float32,
        )

        @pl.when(i >= 2)
        def _reclaim():
            _out_copy(oslot, (i - 2) * TILE_N).wait()

        obuf[oslot] = acc + b_ref[:, pl.ds(s, TILE_N)]
        _out_copy(oslot, s).start()

        @pl.when(i + DEPTH < N_FULL)
        def _refill():
            _w_fill((i + DEPTH) * TILE_N, slot).start()

        @pl.when(i == ISSUE_LAST_AT)
        def _refill_last():
            _w_fill_last().start()

        return 0

    lax.fori_loop(0, N_FULL, tile_step, 0, unroll=4)

    _w_fill_last().wait()
    acc = lax.dot_general(
        h, wbuf_last[...],
        dimension_numbers=(((1,), (1,)), ((), ())),
        preferred_element_type=jnp.float32,
    )
    oslot = N_FULL % 2
    _out_copy(oslot, (N_FULL - 2) * TILE_N).wait()
    obuf_last[...] = acc + b_ref[:, pl.ds(N_FULL * TILE_N, REM)]
    last_out = pltpu.make_async_copy(
        obuf_last, out_hbm.at[:, pl.ds(N_FULL * TILE_N, REM)], osem.at[oslot])
    last_out.start()

    _out_copy(1 - oslot, (N_FULL - 1) * TILE_N).wait()
    last_out.wait()


@jax.jit
def kernel(x, emb_table, W, b):
    B, L = x.shape
    T = B * L
    V, D = W.shape
    idx = x.reshape(T).astype(jnp.int32)
    b2 = b.reshape(1, V)

    logits = pl.pallas_call(
        _body,
        grid_spec=pltpu.PrefetchScalarGridSpec(
            num_scalar_prefetch=1,
            grid=(1,),
            in_specs=[
                pl.BlockSpec(memory_space=pl.ANY),
                pl.BlockSpec(memory_space=pl.ANY),
                pl.BlockSpec(memory_space=pl.ANY),
            ],
            out_specs=pl.BlockSpec(memory_space=pl.ANY),
            scratch_shapes=[
                pltpu.VMEM((T, D), jnp.float32),
                pltpu.VMEM((DEPTH, TILE_N, D), jnp.float32),
                pltpu.VMEM((2, T, TILE_N), jnp.float32),
                pltpu.VMEM((REM, D), jnp.float32),
                pltpu.VMEM((T, REM), jnp.float32),
                pltpu.VMEM((1, V), jnp.float32),
                pltpu.SemaphoreType.DMA,
                pltpu.SemaphoreType.DMA,
                pltpu.SemaphoreType.DMA((DEPTH,)),
                pltpu.SemaphoreType.DMA((2,)),
            ],
        ),
        out_shape=jax.ShapeDtypeStruct((T, V), jnp.float32),
        compiler_params=pltpu.CompilerParams(
            dimension_semantics=("arbitrary",),
        ),
    )(idx, emb_table, W, b2)

    return logits.reshape(B, L, V)


# speedup vs baseline: 3.6101x; 1.0044x over previous
import jax
import jax.numpy as jnp
from jax import lax
from jax.experimental import pallas as pl
from jax.experimental.pallas import tpu as pltpu

DIM = 1024
TILE_N = 2048
DEPTH = 3
NUM_TOKENS = 256
VOCAB = 100000
N_FULL = VOCAB // TILE_N
REM = VOCAB - N_FULL * TILE_N
ISSUE_LAST_AT = N_FULL - DEPTH


def _body(idx_ref, emb_hbm, w_hbm, b_hbm, out_hbm, h_ref, wbuf, obuf,
          wbuf_last, obuf_last, b_ref, gsem, bsem, wsem, osem):
    def _w_fill(j_start, slot):
        return pltpu.make_async_copy(
            w_hbm.at[pl.ds(j_start, TILE_N), :], wbuf.at[slot], wsem.at[slot])

    def _w_fill_last():
        return pltpu.make_async_copy(
            w_hbm.at[pl.ds(N_FULL * TILE_N, REM), :], wbuf_last, gsem)

    def _out_copy(slot, col_start):
        return pltpu.make_async_copy(
            obuf.at[slot], out_hbm.at[:, pl.ds(col_start, TILE_N)],
            osem.at[slot])

    def issue_row(t, _):
        pltpu.make_async_copy(
            emb_hbm.at[pl.ds(idx_ref[t], 1), :],
            h_ref.at[pl.ds(t, 1), :], gsem).start()
        return 0

    def wait_row(t, _):
        pltpu.make_async_copy(
            emb_hbm.at[pl.ds(idx_ref[t], 1), :],
            h_ref.at[pl.ds(t, 1), :], gsem).wait()
        return 0

    lax.fori_loop(0, NUM_TOKENS, issue_row, 0)
    b_copy = pltpu.make_async_copy(b_hbm, b_ref, bsem)
    b_copy.start()
    for j in range(DEPTH):
        _w_fill(j * TILE_N, j).start()
    lax.fori_loop(0, NUM_TOKENS, wait_row, 0)
    b_copy.wait()
    h = h_ref[...]

    def tile_step(i, _):
        slot = lax.rem(i, DEPTH)
        oslot = lax.rem(i, 2)
        s = i * TILE_N
        _w_fill(s, slot).wait()
        acc = lax.dot_general(
            h, wbuf[slot],
            dimension_numbers=(((1,), (1,)), ((), ())),
            preferred_element_type=jnp.float32,
        )

        @pl.when(i >= 2)
        def _reclaim():
            _out_copy(oslot, (i - 2) * TILE_N).wait()

        obuf[oslot] = acc + b_ref[:, pl.ds(s, TILE_N)]
        _out_copy(oslot, s).start()

        @pl.when(i + DEPTH < N_FULL)
        def _refill():
            _w_fill((i + DEPTH) * TILE_N, slot).start()

        @pl.when(i == ISSUE_LAST_AT)
        def _refill_last():
            _w_fill_last().start()

        return 0

    lax.fori_loop(0, N_FULL, tile_step, 0, unroll=8)

    _w_fill_last().wait()
    acc = lax.dot_general(
        h, wbuf_last[...],
        dimension_numbers=(((1,), (1,)), ((), ())),
        preferred_element_type=jnp.float32,
    )
    oslot = N_FULL % 2
    _out_copy(oslot, (N_FULL - 2) * TILE_N).wait()
    obuf_last[...] = acc + b_ref[:, pl.ds(N_FULL * TILE_N, REM)]
    last_out = pltpu.make_async_copy(
        obuf_last, out_hbm.at[:, pl.ds(N_FULL * TILE_N, REM)], osem.at[oslot])
    last_out.start()

    _out_copy(1 - oslot, (N_FULL - 1) * TILE_N).wait()
    last_out.wait()


@jax.jit
def kernel(x, emb_table, W, b):
    B, L = x.shape
    T = B * L
    V, D = W.shape
    idx = x.reshape(T).astype(jnp.int32)
    b2 = b.reshape(1, V)

    logits = pl.pallas_call(
        _body,
        grid_spec=pltpu.PrefetchScalarGridSpec(
            num_scalar_prefetch=1,
            grid=(1,),
            in_specs=[
                pl.BlockSpec(memory_space=pl.ANY),
                pl.BlockSpec(memory_space=pl.ANY),
                pl.BlockSpec(memory_space=pl.ANY),
            ],
            out_specs=pl.BlockSpec(memory_space=pl.ANY),
            scratch_shapes=[
                pltpu.VMEM((T, D), jnp.float32),
                pltpu.VMEM((DEPTH, TILE_N, D), jnp.float32),
                pltpu.VMEM((2, T, TILE_N), jnp.float32),
                pltpu.VMEM((REM, D), jnp.float32),
                pltpu.VMEM((T, REM), jnp.float32),
                pltpu.VMEM((1, V), jnp.float32),
                pltpu.SemaphoreType.DMA,
                pltpu.SemaphoreType.DMA,
                pltpu.SemaphoreType.DMA((DEPTH,)),
                pltpu.SemaphoreType.DMA((2,)),
            ],
        ),
        out_shape=jax.ShapeDtypeStruct((T, V), jnp.float32),
        compiler_params=pltpu.CompilerParams(
            dimension_semantics=("arbitrary",),
        ),
    )(idx, emb_table, W, b2)

    return logits.reshape(B, L, V)


# speedup vs baseline: 3.6171x; 1.0019x over previous
import jax
import jax.numpy as jnp
from jax import lax
from jax.experimental import pallas as pl
from jax.experimental.pallas import tpu as pltpu

DIM = 1024
TILE_N = 2048
DEPTH = 3
NUM_TOKENS = 256
VOCAB = 100000
N_FULL = VOCAB // TILE_N
REM = VOCAB - N_FULL * TILE_N
ISSUE_LAST_AT = N_FULL - DEPTH


def _body(idx_ref, emb_hbm, w_hbm, b_hbm, out_hbm, h_ref, wbuf, obuf,
          wbuf_last, obuf_last, b_ref, gsem, bsem, wsem, osem):
    def _w_fill(j_start, slot):
        return pltpu.make_async_copy(
            w_hbm.at[pl.ds(j_start, TILE_N), :], wbuf.at[slot], wsem.at[slot])

    def _w_fill_last():
        return pltpu.make_async_copy(
            w_hbm.at[pl.ds(N_FULL * TILE_N, REM), :], wbuf_last, gsem)

    def _out_copy(slot, col_start):
        return pltpu.make_async_copy(
            obuf.at[slot], out_hbm.at[:, pl.ds(col_start, 2 * TILE_N)],
            osem.at[slot])

    def issue_row(t, _):
        pltpu.make_async_copy(
            emb_hbm.at[pl.ds(idx_ref[t], 1), :],
            h_ref.at[pl.ds(t, 1), :], gsem).start()
        return 0

    def wait_row(t, _):
        pltpu.make_async_copy(
            emb_hbm.at[pl.ds(idx_ref[t], 1), :],
            h_ref.at[pl.ds(t, 1), :], gsem).wait()
        return 0

    lax.fori_loop(0, NUM_TOKENS, issue_row, 0)
    b_copy = pltpu.make_async_copy(b_hbm, b_ref, bsem)
    b_copy.start()
    for j in range(DEPTH):
        _w_fill(j * TILE_N, j).start()
    lax.fori_loop(0, NUM_TOKENS, wait_row, 0)
    b_copy.wait()
    h = h_ref[...]

    def tile_step(i, _):
        slot = lax.rem(i, DEPTH)
        pair = lax.div(i, 2)
        half = lax.rem(i, 2)
        oslot = lax.rem(pair, 2)
        s = i * TILE_N
        _w_fill(s, slot).wait()
        acc = lax.dot_general(
            h, wbuf[slot],
            dimension_numbers=(((1,), (1,)), ((), ())),
            preferred_element_type=jnp.float32,
        )

        @pl.when((half == 0) & (pair >= 2))
        def _reclaim():
            _out_copy(oslot, (pair - 2) * 2 * TILE_N).wait()

        obuf[oslot, :, pl.ds(half * TILE_N, TILE_N)] = (
            acc + b_ref[:, pl.ds(s, TILE_N)])

        @pl.when(half == 1)
        def _flush():
            _out_copy(oslot, pair * 2 * TILE_N).start()

        @pl.when(i + DEPTH < N_FULL)
        def _refill():
            _w_fill((i + DEPTH) * TILE_N, slot).start()

        @pl.when(i == ISSUE_LAST_AT)
        def _refill_last():
            _w_fill_last().start()

        return 0

    lax.fori_loop(0, N_FULL, tile_step, 0, unroll=8)

    _w_fill_last().wait()
    acc = lax.dot_general(
        h, wbuf_last[...],
        dimension_numbers=(((1,), (1,)), ((), ())),
        preferred_element_type=jnp.float32,
    )
    n_pairs = N_FULL // 2
    obuf_last[...] = acc + b_ref[:, pl.ds(N_FULL * TILE_N, REM)]
    last_out = pltpu.make_async_copy(
        obuf_last, out_hbm.at[:, pl.ds(N_FULL * TILE_N, REM)], gsem)
    last_out.start()

    _out_copy((n_pairs - 2) % 2, (n_pairs - 2) * 2 * TILE_N).wait()
    _out_copy((n_pairs - 1) % 2, (n_pairs - 1) * 2 * TILE_N).wait()
    last_out.wait()


@jax.jit
def kernel(x, emb_table, W, b):
    B, L = x.shape
    T = B * L
    V, D = W.shape
    idx = x.reshape(T).astype(jnp.int32)
    b2 = b.reshape(1, V)

    logits = pl.pallas_call(
        _body,
        grid_spec=pltpu.PrefetchScalarGridSpec(
            num_scalar_prefetch=1,
            grid=(1,),
            in_specs=[
                pl.BlockSpec(memory_space=pl.ANY),
                pl.BlockSpec(memory_space=pl.ANY),
                pl.BlockSpec(memory_space=pl.ANY),
            ],
            out_specs=pl.BlockSpec(memory_space=pl.ANY),
            scratch_shapes=[
                pltpu.VMEM((T, D), jnp.float32),
                pltpu.VMEM((DEPTH, TILE_N, D), jnp.float32),
                pltpu.VMEM((2, T, 2 * TILE_N), jnp.float32),
                pltpu.VMEM((REM, D), jnp.float32),
                pltpu.VMEM((T, REM), jnp.float32),
                pltpu.VMEM((1, V), jnp.float32),
                pltpu.SemaphoreType.DMA,
                pltpu.SemaphoreType.DMA,
                pltpu.SemaphoreType.DMA((DEPTH,)),
                pltpu.SemaphoreType.DMA((2,)),
            ],
        ),
        out_shape=jax.ShapeDtypeStruct((T, V), jnp.float32),
        compiler_params=pltpu.CompilerParams(
            dimension_semantics=("arbitrary",),
        ),
    )(idx, emb_table, W, b2)

    return logits.reshape(B, L, V)
